# baseline (speedup 1.0000x reference)
import functools

import jax
import jax.numpy as jnp
from jax import lax
from jax.experimental import pallas as pl
from jax.experimental.pallas import tpu as pltpu
from jax.experimental.pallas import tpu_sc as plsc

NUM_FIELDS = 26
VOCAB = 100000
EMB = 32
B = 16384
H0 = 256
H1 = 128

FH = NUM_FIELDS // 2
LANES = 16
NC = 2
NS = 16
NW = NC * NS
BPW = B // NW
RPG = 4
IDXPG = RPG * FH
NGROUP = BPW // RPG
NBUF = 4
NBLK = NGROUP // NBUF
LPAD = 64


def _sc_body(widx_hbm, lane_hbm, w2_hbm, fm_hbm, widx_all, lane_all,
             rows_v, out_v, sem0, sem1, sem2, sem3):
    sems = (sem0, sem1, sem2, sem3)
    wid = lax.axis_index("c") * NS + lax.axis_index("s")
    pltpu.sync_copy(widx_hbm.at[pl.ds(wid * NGROUP, NGROUP)], widx_all)
    pltpu.sync_copy(lane_hbm.at[pl.ds(wid * NGROUP, NGROUP)], lane_all)

    def issue(g, b):
        pltpu.async_copy(w2_hbm.at[widx_all.at[g]], rows_v.at[b], sems[b])

    for b in range(NBUF):
        issue(b, b)

    def blk_body(blk, carry):
        for b in range(NBUF):
            g = blk * NBUF + b
            pltpu.make_async_copy(
                w2_hbm.at[widx_all.at[g]], rows_v.at[b], sems[b]).wait()
            lv = [lane_all[g, pl.ds(i * LANES, LANES)]
                  for i in range(LPAD // LANES)]
            for r in range(RPG):
                acc0 = jnp.zeros((LANES,), jnp.float32)
                acc1 = jnp.zeros((LANES,), jnp.float32)
                sq0 = jnp.zeros((LANES,), jnp.float32)
                sq1 = jnp.zeros((LANES,), jnp.float32)
                for k in range(FH):
                    j = r * FH + k
                    lane0 = lv[j // LANES][j % LANES]
                    x0 = rows_v[b, j, pl.ds(lane0, LANES)]
                    x1 = rows_v[b, j, pl.ds(lane0 + LANES, LANES)]
                    acc0 = acc0 + x0
                    sq0 = sq0 + x0 * x0
                    acc1 = acc1 + x1
                    sq1 = sq1 + x1 * x1
                o = b * RPG + r
                out_v[o, pl.ds(0, LANES)] = acc0
                out_v[o, pl.ds(LANES, LANES)] = acc1
                out_v[o, pl.ds(2 * LANES, LANES)] = sq0
                out_v[o, pl.ds(3 * LANES, LANES)] = sq1
            @pl.when(g + NBUF < NGROUP)
            def _():
                issue(g + NBUF, b)
        pltpu.sync_copy(
            out_v, fm_hbm.at[pl.ds(wid * BPW + blk * (NBUF * RPG), NBUF * RPG)])
        return carry

    lax.fori_loop(0, NBLK, blk_body, 0)


_sc_fm = pl.kernel(
    _sc_body,
    out_type=jax.ShapeDtypeStruct((B, 2 * EMB), jnp.float32),
    mesh=plsc.VectorSubcoreMesh(core_axis_name="c", subcore_axis_name="s"),
    scratch_types=[
        pltpu.VMEM((NGROUP, IDXPG), jnp.int32),
        pltpu.VMEM((NGROUP, LPAD), jnp.int32),
        pltpu.VMEM((NBUF, IDXPG, 128), jnp.float32),
        pltpu.VMEM((NBUF * RPG, 2 * EMB), jnp.float32),
        pltpu.SemaphoreType.DMA,
        pltpu.SemaphoreType.DMA,
        pltpu.SemaphoreType.DMA,
        pltpu.SemaphoreType.DMA,
    ],
    compiler_params=pltpu.CompilerParams(use_tc_tiling_on_sc=True),
)

VCH = 102400
NST = VCH // 512
NVC = -(-VOCAB // 512)
NDC = -(-VOCAB // VCH)


def _detile_body(w2t_ref, out_ref):
    for s in range(NST):
        x = w2t_ref[0, :, pl.ds(s * 512, 512)]
        xs = jnp.concatenate([x[:, k * 128:(k + 1) * 128] for k in range(4)],
                             axis=0)
        out_ref[0, pl.ds(s * 128, 128), :] = jnp.transpose(xs)


def _make_detile(f0):
    return pl.pallas_call(
        _detile_body,
        grid=(FH, NDC),
        in_specs=[pl.BlockSpec((1, EMB, VCH), lambda f, c: (f + f0, 0, c))],
        out_specs=pl.BlockSpec((1, VCH // 4, 128), lambda f, c: (f, c, 0)),
        out_shape=jax.ShapeDtypeStruct((FH, NVC * 128, 128), jnp.float32),
    )


_detile_a = _make_detile(0)
_detile_b = _make_detile(FH)

BS = 2048


def _dnn_body(pa_ref, pb_ref, wh0_ref, bh0_ref, wh1_ref, bh1_ref, wl_ref,
              bl_ref, bias_ref, out_ref):
    a = pa_ref[...]
    b = pb_ref[...]
    s = a[:, :EMB] + b[:, :EMB]
    q = a[:, EMB:] + b[:, EMB:]
    x = (s * s - q) * 0.005
    h = jax.nn.sigmoid(
        jnp.dot(x, wh0_ref[...], preferred_element_type=jnp.float32)
        + bh0_ref[...][None, :])
    h = jax.nn.sigmoid(
        jnp.dot(h, wh1_ref[...], preferred_element_type=jnp.float32)
        + bh1_ref[...][None, :])
    deep = jnp.sum(h * wl_ref[...][:, 0][None, :], axis=1)
    total = deep + jnp.sum(x, axis=1) + (bl_ref[...] + bias_ref[...])
    out_ref[...] = total[:, None]


_dnn = pl.pallas_call(
    _dnn_body,
    grid=(B // BS,),
    in_specs=[
        pl.BlockSpec((BS, 2 * EMB), lambda i: (i, 0)),
        pl.BlockSpec((BS, 2 * EMB), lambda i: (i, 0)),
        pl.BlockSpec((EMB, H0), lambda i: (0, 0)),
        pl.BlockSpec((H0,), lambda i: (0,)),
        pl.BlockSpec((H0, H1), lambda i: (0, 0)),
        pl.BlockSpec((H1,), lambda i: (0,)),
        pl.BlockSpec((H1, 1), lambda i: (0, 0)),
        pl.BlockSpec((1,), lambda i: (0,)),
        pl.BlockSpec((1,), lambda i: (0,)),
    ],
    out_specs=pl.BlockSpec((BS, 1), lambda i: (i, 0)),
    out_shape=jax.ShapeDtypeStruct((B, 1), jnp.float32),
)


def _half_indices(cath):
    fbase = (jnp.arange(FH, dtype=jnp.int32) * (NVC * 128))[None, :]
    widx = (fbase + ((cath >> 9) << 7) + (cath & 127)).reshape(B // RPG, IDXPG)
    lane0 = (((cath >> 7) & 3) << 5).reshape(B // RPG, IDXPG)
    lane0 = jnp.pad(lane0, ((0, 0), (0, LPAD - IDXPG)))
    return widx, lane0


def kernel(cat_feat, flag, W1, W2, Wh0, bh0, Wh1, bh1, Wl, bl, bias):
    cat = cat_feat.astype(jnp.int32)
    widx_a, lane_a = _half_indices(cat[:, :FH])
    widx_b, lane_b = _half_indices(cat[:, FH:])
    w2t = jnp.transpose(W2, (0, 2, 1))
    widx_a, lane_a, widx_b, lane_b, w2t = lax.optimization_barrier(
        (widx_a, lane_a, widx_b, lane_b, w2t))
    wide_a = _detile_a(w2t).reshape(FH * NVC * 128, 128)
    wide_b = _detile_b(w2t).reshape(FH * NVC * 128, 128)
    pa = _sc_fm(widx_a, lane_a, wide_a)
    pb = _sc_fm(widx_b, lane_b, wide_b)
    return _dnn(pa, pb, Wh0, bh0, Wh1, bh1, Wl, bl, bias)


# speedup vs baseline: 1.0834x; 1.0834x over previous
import functools

import jax
import jax.numpy as jnp
from jax import lax
from jax.experimental import pallas as pl
from jax.experimental.pallas import tpu as pltpu
from jax.experimental.pallas import tpu_sc as plsc

NUM_FIELDS = 26
VOCAB = 100000
EMB = 32
B = 16384
H0 = 256
H1 = 128

LANES = 16
NC = 2
NS = 16
NW = NC * NS
BPW = B // NW
RPG = 4
IDXPG = RPG * NUM_FIELDS
NGROUP = BPW // RPG
NBUF = 4
NBLK = NGROUP // NBUF
WROWS = NUM_FIELDS * VOCAB // 4
LPAD = 112


def _sc_body(widx_hbm, lane_hbm, w2_hbm, fm2_hbm, widx_all, lane_all,
             rows_v, out_v, sem0, sem1, sem2, sem3):
    sems = (sem0, sem1, sem2, sem3)
    wid = lax.axis_index("c") * NS + lax.axis_index("s")
    pltpu.sync_copy(widx_hbm.at[pl.ds(wid * NGROUP, NGROUP)], widx_all)
    pltpu.sync_copy(lane_hbm.at[pl.ds(wid * NGROUP, NGROUP)], lane_all)

    def issue(g, b):
        pltpu.async_copy(w2_hbm.at[widx_all.at[g]], rows_v.at[b], sems[b])

    for b in range(NBUF):
        issue(b, b)

    def blk_body(blk, carry):
        for b in range(NBUF):
            g = blk * NBUF + b
            pltpu.make_async_copy(
                w2_hbm.at[widx_all.at[g]], rows_v.at[b], sems[b]).wait()
            lv = [lane_all[g, pl.ds(i * LANES, LANES)] for i in range(LPAD // LANES)]
            for r in range(RPG):
                acc0 = jnp.zeros((LANES,), jnp.float32)
                acc1 = jnp.zeros((LANES,), jnp.float32)
                sq0 = jnp.zeros((LANES,), jnp.float32)
                sq1 = jnp.zeros((LANES,), jnp.float32)
                for k in range(NUM_FIELDS):
                    j = r * NUM_FIELDS + k
                    lane0 = lv[j // LANES][j % LANES]
                    x0 = rows_v[b, j, pl.ds(lane0, LANES)]
                    x1 = rows_v[b, j, pl.ds(lane0 + LANES, LANES)]
                    acc0 = acc0 + x0
                    sq0 = sq0 + x0 * x0
                    acc1 = acc1 + x1
                    sq1 = sq1 + x1 * x1
                out_v[b * RPG + r, pl.ds(0, LANES)] = (acc0 * acc0 - sq0) * 0.005
                out_v[b * RPG + r, pl.ds(LANES, LANES)] = (acc1 * acc1 - sq1) * 0.005
            @pl.when(g + NBUF < NGROUP)
            def _():
                issue(g + NBUF, b)
        pltpu.sync_copy(
            out_v, fm2_hbm.at[pl.ds(wid * BPW + blk * (NBUF * RPG), NBUF * RPG)])
        return carry

    lax.fori_loop(0, NBLK, blk_body, 0)


_sc_fm = pl.kernel(
    _sc_body,
    out_type=jax.ShapeDtypeStruct((B, EMB), jnp.float32),
    mesh=plsc.VectorSubcoreMesh(core_axis_name="c", subcore_axis_name="s"),
    scratch_types=[
        pltpu.VMEM((NGROUP, IDXPG), jnp.int32),
        pltpu.VMEM((NGROUP, LPAD), jnp.int32),
        pltpu.VMEM((NBUF, IDXPG, 128), jnp.float32),
        pltpu.VMEM((NBUF * RPG, EMB), jnp.float32),
        pltpu.SemaphoreType.DMA,
        pltpu.SemaphoreType.DMA,
        pltpu.SemaphoreType.DMA,
        pltpu.SemaphoreType.DMA,
    ],
    compiler_params=pltpu.CompilerParams(use_tc_tiling_on_sc=True),
)

VCH = 102400
NST = VCH // 512
NVC = -(-VOCAB // 512)
NDC = -(-VOCAB // VCH)


def _detile_body(w2t_ref, out_ref):
    for s in range(NST):
        x = w2t_ref[0, :, pl.ds(s * 512, 512)]
        xs = jnp.concatenate([x[:, k * 128:(k + 1) * 128] for k in range(4)],
                             axis=0)
        out_ref[0, pl.ds(s * 128, 128), :] = jnp.transpose(xs)


_detile = pl.pallas_call(
    _detile_body,
    grid=(NUM_FIELDS, NDC),
    in_specs=[pl.BlockSpec((1, EMB, VCH), lambda f, c: (f, 0, c))],
    out_specs=pl.BlockSpec((1, VCH // 4, 128), lambda f, c: (f, c, 0)),
    out_shape=jax.ShapeDtypeStruct((NUM_FIELDS, NVC * 128, 128), jnp.float32),
)

BS = 2048


def _dnn_body(fm_ref, wh0_ref, bh0_ref, wh1_ref, bh1_ref, wl_ref, bl_ref,
              bias_ref, out_ref):
    x = fm_ref[...]
    h = jax.nn.sigmoid(
        jnp.dot(x, wh0_ref[...], preferred_element_type=jnp.float32)
        + bh0_ref[...][None, :])
    h = jax.nn.sigmoid(
        jnp.dot(h, wh1_ref[...], preferred_element_type=jnp.float32)
        + bh1_ref[...][None, :])
    deep = jnp.sum(h * wl_ref[...][:, 0][None, :], axis=1)
    total = deep + jnp.sum(x, axis=1) + (bl_ref[...] + bias_ref[...])
    out_ref[...] = total[:, None]


_dnn = pl.pallas_call(
    _dnn_body,
    grid=(B // BS,),
    in_specs=[
        pl.BlockSpec((BS, EMB), lambda i: (i, 0)),
        pl.BlockSpec((EMB, H0), lambda i: (0, 0)),
        pl.BlockSpec((H0,), lambda i: (0,)),
        pl.BlockSpec((H0, H1), lambda i: (0, 0)),
        pl.BlockSpec((H1,), lambda i: (0,)),
        pl.BlockSpec((H1, 1), lambda i: (0, 0)),
        pl.BlockSpec((1,), lambda i: (0,)),
        pl.BlockSpec((1,), lambda i: (0,)),
    ],
    out_specs=pl.BlockSpec((BS, 1), lambda i: (i, 0)),
    out_shape=jax.ShapeDtypeStruct((B, 1), jnp.float32),
)


def kernel(cat_feat, flag, W1, W2, Wh0, bh0, Wh1, bh1, Wl, bl, bias):
    cat = cat_feat.astype(jnp.int32)
    fbase = (jnp.arange(NUM_FIELDS, dtype=jnp.int32) * (NVC * 128))[None, :]
    widx = (fbase + ((cat >> 9) << 7) + (cat & 127)).reshape(B // RPG, IDXPG)
    lane0 = (((cat >> 7) & 3) << 5).reshape(B // RPG, IDXPG)
    lane0 = jnp.pad(lane0, ((0, 0), (0, LPAD - IDXPG)))
    w2_wide = _detile(jnp.transpose(W2, (0, 2, 1))).reshape(NUM_FIELDS * NVC * 128, 128)
    fm2 = _sc_fm(widx, lane0, w2_wide)
    return _dnn(fm2, Wh0, bh0, Wh1, bh1, Wl, bl, bias)
